# baseline (speedup 1.0000x reference)
import functools

import jax
import jax.numpy as jnp
from jax import lax
from jax.experimental import pallas as pl
from jax.experimental.pallas import tpu as pltpu
from jax.experimental.pallas import tpu_sc as plsc

VOCAB = 1000000
EMBED = 64
BATCH = 16384
NEG = 10

_NC = 2
_NS = 16
_NW = _NC * _NS
_LANES = 16

_BPW = BATCH // _NW
_CH = 64
_NCHUNK = _BPW // _CH
_NGRP = _CH // _LANES


def _sc_scores():
    mesh = plsc.VectorSubcoreMesh(core_axis_name="c", subcore_axis_name="s")

    @functools.partial(
        pl.kernel,
        mesh=mesh,
        compiler_params=pltpu.CompilerParams(needs_layout_passes=False,
                                             use_tc_tiling_on_sc=True),
        out_type=jax.ShapeDtypeStruct(((1 + NEG) * BATCH,), jnp.float32),
        scratch_types=[
            pltpu.VMEM((_CH,), jnp.int32),
            pltpu.VMEM((_CH,), jnp.int32),
            pltpu.VMEM((NEG * _CH,), jnp.int32),
            pltpu.VMEM((_CH, EMBED), jnp.float32),
            pltpu.VMEM((_CH, EMBED), jnp.float32),
            pltpu.VMEM((NEG * _CH, EMBED), jnp.float32),
            pltpu.VMEM(((1 + NEG) * _CH,), jnp.float32),
            pltpu.SemaphoreType.DMA,
        ],
    )
    def sc_scores(iw_hbm, cw_hbm, neg_hbm, win_hbm, wctx_hbm, out_hbm,
                  iidx, cidx, nidx, irows, crows, nrows, scores, sem):
        wid = lax.axis_index("s") * _NC + lax.axis_index("c")
        lane0 = lax.iota(jnp.int32, _LANES) == 0

        def chunk_body(j, _):
            base = wid * _BPW + j * _CH
            pltpu.sync_copy(iw_hbm.at[pl.ds(base, _CH)], iidx)
            pltpu.sync_copy(cw_hbm.at[pl.ds(base, _CH)], cidx)
            pltpu.sync_copy(neg_hbm.at[pl.ds(base * NEG, NEG * _CH)], nidx)

            def fire_body(g, _):
                goff = g * _LANES
                iv = iidx[pl.ds(goff, _LANES)]
                cv = cidx[pl.ds(goff, _LANES)]
                for j16 in range(_LANES):
                    l = goff + j16
                    si = lax.index_in_dim(iv, j16, 0, keepdims=False)
                    pltpu.async_copy(win_hbm.at[pl.ds(si, 1), :],
                                     irows.at[pl.ds(l, 1), :], sem)
                    sc = lax.index_in_dim(cv, j16, 0, keepdims=False)
                    pltpu.async_copy(wctx_hbm.at[pl.ds(sc, 1), :],
                                     crows.at[pl.ds(l, 1), :], sem)
                for k in range(NEG):
                    nv = nidx[pl.ds(k * _CH + goff, _LANES)]
                    for j16 in range(_LANES):
                        sn = lax.index_in_dim(nv, j16, 0, keepdims=False)
                        pltpu.async_copy(
                            wctx_hbm.at[pl.ds(sn, 1), :],
                            nrows.at[pl.ds(k * _CH + goff + j16, 1), :], sem)
                return 0

            lax.fori_loop(0, _NGRP, fire_body, 0)
            pltpu.make_async_copy(win_hbm.at[pl.ds(0, _CH), :], irows,
                                  sem).wait()
            pltpu.make_async_copy(win_hbm.at[pl.ds(0, _CH), :], crows,
                                  sem).wait()
            pltpu.make_async_copy(win_hbm.at[pl.ds(0, NEG * _CH), :], nrows,
                                  sem).wait()

            def elem_body(b, _):
                vin = [irows[b, pl.ds(q * _LANES, _LANES)] for q in range(4)]
                vctx = [crows[b, pl.ds(q * _LANES, _LANES)] for q in range(4)]
                acc = vin[0] * vctx[0]
                for q in range(1, 4):
                    acc = acc + vin[q] * vctx[q]
                s = jnp.sum(acc)
                plsc.store_scatter(scores, [jnp.full((_LANES,), b, jnp.int32)],
                                   jnp.full((_LANES,), s, jnp.float32),
                                   mask=lane0)
                for k in range(NEG):
                    vng = [nrows[k * _CH + b, pl.ds(q * _LANES, _LANES)]
                           for q in range(4)]
                    nacc = vin[0] * vng[0]
                    for q in range(1, 4):
                        nacc = nacc + vin[q] * vng[q]
                    ns = -jnp.sum(nacc)
                    plsc.store_scatter(
                        scores,
                        [jnp.full((_LANES,), (1 + k) * _CH + b, jnp.int32)],
                        jnp.full((_LANES,), ns, jnp.float32),
                        mask=lane0)
                return 0

            lax.fori_loop(0, _CH, elem_body, 0)
            for k in range(1 + NEG):
                pltpu.sync_copy(scores.at[pl.ds(k * _CH, _CH)],
                                out_hbm.at[pl.ds(k * BATCH + base, _CH)])
            return 0

        lax.fori_loop(0, _NCHUNK, chunk_body, 0)

    return sc_scores


_SC_SCORES = _sc_scores()

_TBLK = 512


def _transpose_body(t_ref, o_ref):
    o_ref[...] = t_ref[...].T


def _to_row_major(w_t):
    grid = (VOCAB + _TBLK - 1) // _TBLK
    return pl.pallas_call(
        _transpose_body,
        grid=(grid,),
        in_specs=[pl.BlockSpec((EMBED, _TBLK), lambda g: (0, g))],
        out_specs=pl.BlockSpec((_TBLK, EMBED), lambda g: (g, 0)),
        out_shape=jax.ShapeDtypeStruct((VOCAB, EMBED), jnp.float32),
    )(w_t)


_ROWS = (1 + NEG) * BATCH // 128


def _loss_body(s_ref, o_ref):
    x = s_ref[...]
    ls = jnp.minimum(x, 0.0) - jnp.log1p(jnp.exp(-jnp.abs(x)))
    o_ref[0, 0] = -jnp.sum(ls) / BATCH


def kernel(input_word, context_word, W_in, W_ctx):
    batch_size = context_word.shape[0]
    neg_key = jax.random.key(1234)
    negative_example = jax.random.randint(neg_key, (batch_size, NEG), 0, VOCAB)
    neg_cm = (negative_example.astype(jnp.int32)
              .reshape(batch_size // _CH, _CH, NEG)
              .transpose(0, 2, 1)
              .reshape(-1))

    w_in_rm = _to_row_major(W_in.T)
    w_ctx_rm = _to_row_major(W_ctx.T)
    scores = _SC_SCORES(input_word.astype(jnp.int32),
                        context_word.astype(jnp.int32),
                        neg_cm, w_in_rm, w_ctx_rm)

    loss = pl.pallas_call(
        _loss_body,
        out_shape=jax.ShapeDtypeStruct((1, 1), jnp.float32),
        out_specs=pl.BlockSpec(memory_space=pltpu.SMEM),
    )(scores.reshape(_ROWS, 128))
    return loss[0, 0]


# speedup vs baseline: 2.1497x; 2.1497x over previous
import functools

import jax
import jax.numpy as jnp
from jax import lax
from jax.experimental import pallas as pl
from jax.experimental.pallas import tpu as pltpu
from jax.experimental.pallas import tpu_sc as plsc

VOCAB = 1000000
EMBED = 64
BATCH = 16384
NEG = 10

_NC = 2
_NS = 16
_NW = _NC * _NS
_LANES = 16

_BPW = BATCH // _NW
_CH = 64
_NCHUNK = _BPW // _CH
_NGRP = _CH // _LANES

_TBLK = 2048
_NBLK = (VOCAB + _TBLK - 1) // _TBLK
_HB = _TBLK // 2
_PR = _NBLK * _HB
_PW = 2 * EMBED


def _sc_scores():
    mesh = plsc.VectorSubcoreMesh(core_axis_name="c", subcore_axis_name="s")

    @functools.partial(
        pl.kernel,
        mesh=mesh,
        compiler_params=pltpu.CompilerParams(needs_layout_passes=False,
                                             use_tc_tiling_on_sc=True),
        out_type=jax.ShapeDtypeStruct(((1 + NEG) * BATCH,), jnp.float32),
        scratch_types=[
            pltpu.VMEM((_CH,), jnp.int32),
            pltpu.VMEM((_CH,), jnp.int32),
            pltpu.VMEM((NEG * _CH,), jnp.int32),
            pltpu.VMEM((_CH,), jnp.int32),
            pltpu.VMEM((_CH,), jnp.int32),
            pltpu.VMEM((NEG * _CH,), jnp.int32),
            pltpu.VMEM((_CH, _PW), jnp.float32),
            pltpu.VMEM((_CH, _PW), jnp.float32),
            pltpu.VMEM((NEG * _CH, _PW), jnp.float32),
            pltpu.VMEM(((1 + NEG) * _CH,), jnp.float32),
            pltpu.SemaphoreType.DMA,
        ],
    )
    def sc_scores(iw_hbm, cw_hbm, neg_hbm, win_hbm, wctx_hbm, out_hbm,
                  iidx, cidx, nidx, ipar, cpar, npar,
                  irows, crows, nrows, scores, sem):
        wid = lax.axis_index("s") * _NC + lax.axis_index("c")
        lane0 = lax.iota(jnp.int32, _LANES) == 0

        def chunk_body(j, _):
            base = wid * _BPW + j * _CH
            pltpu.sync_copy(iw_hbm.at[pl.ds(base, _CH)], iidx)
            pltpu.sync_copy(cw_hbm.at[pl.ds(base, _CH)], cidx)
            pltpu.sync_copy(neg_hbm.at[pl.ds(base * NEG, NEG * _CH)], nidx)

            def split_body(t, _):
                off = t * _LANES
                for buf, par in ((iidx, ipar), (cidx, cpar)):
                    v = buf[pl.ds(off, _LANES)]
                    par[pl.ds(off, _LANES)] = ((v >> 10) & 1) << 6
                    buf[pl.ds(off, _LANES)] = ((v >> 11) << 10) | (v & 1023)
                return 0

            lax.fori_loop(0, _NGRP, split_body, 0)

            def nsplit_body(t, _):
                off = t * _LANES
                v = nidx[pl.ds(off, _LANES)]
                npar[pl.ds(off, _LANES)] = ((v >> 10) & 1) << 6
                nidx[pl.ds(off, _LANES)] = ((v >> 11) << 10) | (v & 1023)
                return 0

            lax.fori_loop(0, NEG * _NGRP, nsplit_body, 0)

            copies = [
                pltpu.async_copy(win_hbm.at[iidx], irows, sem),
                pltpu.async_copy(wctx_hbm.at[cidx], crows, sem),
            ]
            for k in range(NEG):
                copies.append(
                    pltpu.async_copy(wctx_hbm.at[nidx.at[pl.ds(k * _CH, _CH)]],
                                     nrows.at[pl.ds(k * _CH, _CH)], sem))
            for c in copies:
                c.wait()

            def group_body(g, _):
                goff = g * _LANES
                pvi = ipar[pl.ds(goff, _LANES)]
                pvc = cpar[pl.ds(goff, _LANES)]
                pvn = [npar[pl.ds(k * _CH + goff, _LANES)] for k in range(NEG)]
                for j16 in range(_LANES):
                    l = goff + j16
                    hi = lax.index_in_dim(pvi, j16, 0, keepdims=False)
                    hc = lax.index_in_dim(pvc, j16, 0, keepdims=False)
                    vin = [irows[l, pl.ds(hi + q * _LANES, _LANES)]
                           for q in range(4)]
                    vctx = [crows[l, pl.ds(hc + q * _LANES, _LANES)]
                            for q in range(4)]
                    acc = vin[0] * vctx[0]
                    for q in range(1, 4):
                        acc = acc + vin[q] * vctx[q]
                    s = jnp.sum(acc)
                    plsc.store_scatter(
                        scores, [jnp.full((_LANES,), l, jnp.int32)],
                        jnp.full((_LANES,), s, jnp.float32), mask=lane0)
                    for k in range(NEG):
                        hn = lax.index_in_dim(pvn[k], j16, 0, keepdims=False)
                        vng = [nrows[k * _CH + l, pl.ds(hn + q * _LANES, _LANES)]
                               for q in range(4)]
                        nacc = vin[0] * vng[0]
                        for q in range(1, 4):
                            nacc = nacc + vin[q] * vng[q]
                        ns = -jnp.sum(nacc)
                        plsc.store_scatter(
                            scores,
                            [jnp.full((_LANES,), (1 + k) * _CH + l, jnp.int32)],
                            jnp.full((_LANES,), ns, jnp.float32), mask=lane0)
                return 0

            lax.fori_loop(0, _NGRP, group_body, 0)
            for k in range(1 + NEG):
                pltpu.sync_copy(scores.at[pl.ds(k * _CH, _CH)],
                                out_hbm.at[pl.ds(k * BATCH + base, _CH)])
            return 0

        lax.fori_loop(0, _NCHUNK, chunk_body, 0)

    return sc_scores


_SC_SCORES = _sc_scores()

def _pack_body(t_ref, o_ref):
    t = t_ref[...].T
    o_ref[...] = jnp.concatenate([t[:_HB], t[_HB:]], axis=1)


def _to_packed(w_t):
    return pl.pallas_call(
        _pack_body,
        grid=(_NBLK,),
        in_specs=[pl.BlockSpec((EMBED, _TBLK), lambda g: (0, g))],
        out_specs=pl.BlockSpec((_HB, _PW), lambda g: (g, 0)),
        out_shape=jax.ShapeDtypeStruct((_PR, _PW), jnp.float32),
    )(w_t)


_ROWS = (1 + NEG) * BATCH // 128


def _loss_body(s_ref, o_ref):
    x = s_ref[...]
    ls = jnp.minimum(x, 0.0) - jnp.log1p(jnp.exp(-jnp.abs(x)))
    o_ref[0, 0] = -jnp.sum(ls) / BATCH


def kernel(input_word, context_word, W_in, W_ctx):
    batch_size = context_word.shape[0]
    neg_key = jax.random.key(1234)
    negative_example = jax.random.randint(neg_key, (batch_size, NEG), 0, VOCAB)
    neg_cm = (negative_example.astype(jnp.int32)
              .reshape(batch_size // _CH, _CH, NEG)
              .transpose(0, 2, 1)
              .reshape(-1))

    w_in_p = _to_packed(W_in.T)
    w_ctx_p = _to_packed(W_ctx.T)
    scores = _SC_SCORES(input_word.astype(jnp.int32),
                        context_word.astype(jnp.int32),
                        neg_cm, w_in_p, w_ctx_p)

    loss = pl.pallas_call(
        _loss_body,
        out_shape=jax.ShapeDtypeStruct((1, 1), jnp.float32),
        out_specs=pl.BlockSpec(memory_space=pltpu.SMEM),
    )(scores.reshape(_ROWS, 128))
    return loss[0, 0]


# speedup vs baseline: 2.9616x; 1.3777x over previous
import functools

import jax
import jax.numpy as jnp
from jax import lax
from jax.experimental import pallas as pl
from jax.experimental.pallas import tpu as pltpu
from jax.experimental.pallas import tpu_sc as plsc

VOCAB = 1000000
EMBED = 64
BATCH = 16384
NEG = 10

_NC = 2
_NS = 16
_NW = _NC * _NS
_LANES = 16

_BPW = BATCH // _NW
_CH = 32
_NCHUNK = _BPW // _CH
_NGRP = _CH // _LANES

_TBLK = 4096
_NBLK = (VOCAB + _TBLK - 1) // _TBLK
_HB = _TBLK // 2
_PR = _NBLK * _HB
_PW = 2 * EMBED
_SHB = _TBLK.bit_length() - 1
_SHH = _HB.bit_length() - 1


def _sc_scores():
    mesh = plsc.VectorSubcoreMesh(core_axis_name="c", subcore_axis_name="s")

    @functools.partial(
        pl.kernel,
        mesh=mesh,
        compiler_params=pltpu.CompilerParams(needs_layout_passes=False,
                                             use_tc_tiling_on_sc=True),
        out_type=jax.ShapeDtypeStruct(((1 + NEG) * BATCH,), jnp.float32),
        scratch_types=[
            pltpu.VMEM((2 * _CH,), jnp.int32),
            pltpu.VMEM((2 * _CH,), jnp.int32),
            pltpu.VMEM((2 * NEG * _CH,), jnp.int32),
            pltpu.VMEM((2 * _CH,), jnp.int32),
            pltpu.VMEM((2 * _CH,), jnp.int32),
            pltpu.VMEM((2 * NEG * _CH,), jnp.int32),
            pltpu.VMEM((2 * _CH, _PW), jnp.float32),
            pltpu.VMEM((2 * _CH, _PW), jnp.float32),
            pltpu.VMEM((2 * NEG * _CH, _PW), jnp.float32),
            pltpu.VMEM(((1 + NEG) * _CH,), jnp.float32),
            pltpu.SemaphoreType.DMA,
            pltpu.SemaphoreType.DMA,
        ],
    )
    def sc_scores(iw_hbm, cw_hbm, neg_hbm, win_hbm, wctx_hbm, out_hbm,
                  iidx, cidx, nidx, ipar, cpar, npar,
                  irows, crows, nrows, scores, semA, semB):
        wid = lax.axis_index("s") * _NC + lax.axis_index("c")
        lane0 = lax.iota(jnp.int32, _LANES) == 0
        sems = (semA, semB)

        def stage_fire(c):
            p = c & 1
            io, no = p * _CH, p * NEG * _CH
            base = wid * _BPW + c * _CH
            pltpu.sync_copy(iw_hbm.at[pl.ds(base, _CH)],
                            iidx.at[pl.ds(io, _CH)])
            pltpu.sync_copy(cw_hbm.at[pl.ds(base, _CH)],
                            cidx.at[pl.ds(io, _CH)])
            pltpu.sync_copy(neg_hbm.at[pl.ds(base * NEG, NEG * _CH)],
                            nidx.at[pl.ds(no, NEG * _CH)])

            def split_body(t, _):
                off = io + t * _LANES
                for buf, par in ((iidx, ipar), (cidx, cpar)):
                    v = buf[pl.ds(off, _LANES)]
                    par[pl.ds(off, _LANES)] = ((v >> _SHH) & 1) << 6
                    buf[pl.ds(off, _LANES)] = (
                        ((v >> _SHB) << _SHH) | (v & (_HB - 1)))
                return 0

            lax.fori_loop(0, _NGRP, split_body, 0)

            def nsplit_body(t, _):
                off = no + t * _LANES
                v = nidx[pl.ds(off, _LANES)]
                npar[pl.ds(off, _LANES)] = ((v >> _SHH) & 1) << 6
                nidx[pl.ds(off, _LANES)] = (
                    ((v >> _SHB) << _SHH) | (v & (_HB - 1)))
                return 0

            lax.fori_loop(0, NEG * _NGRP, nsplit_body, 0)

            def fire(sem):
                pltpu.async_copy(win_hbm.at[iidx.at[pl.ds(io, _CH)]],
                                 irows.at[pl.ds(io, _CH)], sem)
                pltpu.async_copy(wctx_hbm.at[cidx.at[pl.ds(io, _CH)]],
                                 crows.at[pl.ds(io, _CH)], sem)
                for k in range(NEG):
                    pltpu.async_copy(
                        wctx_hbm.at[nidx.at[pl.ds(no + k * _CH, _CH)]],
                        nrows.at[pl.ds(no + k * _CH, _CH)], sem)

            @pl.when(p == 0)
            def _():
                fire(semA)

            @pl.when(p == 1)
            def _():
                fire(semB)

        def drain(p):
            sem = sems[p]
            io, no = p * _CH, p * NEG * _CH
            pltpu.make_async_copy(win_hbm.at[pl.ds(0, _CH)],
                                  irows.at[pl.ds(io, _CH)], sem).wait()
            pltpu.make_async_copy(win_hbm.at[pl.ds(0, _CH)],
                                  crows.at[pl.ds(io, _CH)], sem).wait()
            pltpu.make_async_copy(win_hbm.at[pl.ds(0, NEG * _CH)],
                                  nrows.at[pl.ds(no, NEG * _CH)], sem).wait()

        def compute_out(c):
            p = c & 1
            io, no = p * _CH, p * NEG * _CH
            base = wid * _BPW + c * _CH

            def group_body(g, _):
                goff = g * _LANES
                pvi = ipar[pl.ds(io + goff, _LANES)]
                pvc = cpar[pl.ds(io + goff, _LANES)]
                pvn = [npar[pl.ds(no + k * _CH + goff, _LANES)]
                       for k in range(NEG)]
                for j16 in range(_LANES):
                    l = goff + j16
                    hi = lax.index_in_dim(pvi, j16, 0, keepdims=False)
                    hc = lax.index_in_dim(pvc, j16, 0, keepdims=False)
                    vin = [irows[io + l, pl.ds(hi + q * _LANES, _LANES)]
                           for q in range(4)]
                    vctx = [crows[io + l, pl.ds(hc + q * _LANES, _LANES)]
                            for q in range(4)]
                    acc = vin[0] * vctx[0]
                    for q in range(1, 4):
                        acc = acc + vin[q] * vctx[q]
                    s = jnp.sum(acc)
                    plsc.store_scatter(
                        scores, [jnp.full((_LANES,), l, jnp.int32)],
                        jnp.full((_LANES,), s, jnp.float32), mask=lane0)
                    for k in range(NEG):
                        hn = lax.index_in_dim(pvn[k], j16, 0, keepdims=False)
                        vng = [nrows[no + k * _CH + l,
                                     pl.ds(hn + q * _LANES, _LANES)]
                               for q in range(4)]
                        nacc = vin[0] * vng[0]
                        for q in range(1, 4):
                            nacc = nacc + vin[q] * vng[q]
                        ns = -jnp.sum(nacc)
                        plsc.store_scatter(
                            scores,
                            [jnp.full((_LANES,), (1 + k) * _CH + l, jnp.int32)],
                            jnp.full((_LANES,), ns, jnp.float32), mask=lane0)
                return 0

            lax.fori_loop(0, _NGRP, group_body, 0)
            for k in range(1 + NEG):
                pltpu.sync_copy(scores.at[pl.ds(k * _CH, _CH)],
                                out_hbm.at[pl.ds(k * BATCH + base, _CH)])

        stage_fire(jnp.int32(0))

        def chunk_body(c, _):
            @pl.when(c + 1 < _NCHUNK)
            def _():
                stage_fire(c + 1)

            @pl.when((c & 1) == 0)
            def _():
                drain(0)

            @pl.when((c & 1) == 1)
            def _():
                drain(1)

            compute_out(c)
            return 0

        lax.fori_loop(0, _NCHUNK, chunk_body, 0)

    return sc_scores


_SC_SCORES = _sc_scores()

def _pack_body(t_ref, o_ref):
    x = t_ref[...]
    x3 = jnp.concatenate([x[:, :_HB], x[:, _HB:]], axis=0)
    eye = jnp.eye(_PW, dtype=jnp.float32)
    o_ref[...] = lax.dot_general(x3, eye, (((0,), (0,)), ((), ())),
                                 preferred_element_type=jnp.float32)


def _to_packed(w_t):
    return pl.pallas_call(
        _pack_body,
        grid=(_NBLK,),
        in_specs=[pl.BlockSpec((EMBED, _TBLK), lambda g: (0, g))],
        out_specs=pl.BlockSpec((_HB, _PW), lambda g: (g, 0)),
        out_shape=jax.ShapeDtypeStruct((_PR, _PW), jnp.float32),
    )(w_t)


_ROWS = (1 + NEG) * BATCH // 128


def _loss_body(s_ref, o_ref):
    x = s_ref[...]
    ls = jnp.minimum(x, 0.0) - jnp.log1p(jnp.exp(-jnp.abs(x)))
    o_ref[0, 0] = -jnp.sum(ls) / BATCH


def kernel(input_word, context_word, W_in, W_ctx):
    batch_size = context_word.shape[0]
    neg_key = jax.random.key(1234)
    negative_example = jax.random.randint(neg_key, (batch_size, NEG), 0, VOCAB)
    neg_cm = (negative_example.astype(jnp.int32)
              .reshape(batch_size // _CH, _CH, NEG)
              .transpose(0, 2, 1)
              .reshape(-1))

    w_in_p = _to_packed(W_in.T)
    w_ctx_p = _to_packed(W_ctx.T)
    scores = _SC_SCORES(input_word.astype(jnp.int32),
                        context_word.astype(jnp.int32),
                        neg_cm, w_in_p, w_ctx_p)

    loss = pl.pallas_call(
        _loss_body,
        out_shape=jax.ShapeDtypeStruct((1, 1), jnp.float32),
        out_specs=pl.BlockSpec(memory_space=pltpu.SMEM),
    )(scores.reshape(_ROWS, 128))
    return loss[0, 0]


# speedup vs baseline: 3.5772x; 1.2079x over previous
import functools

import jax
import jax.numpy as jnp
from jax import lax
from jax.experimental import pallas as pl
from jax.experimental.pallas import tpu as pltpu
from jax.experimental.pallas import tpu_sc as plsc

VOCAB = 1000000
EMBED = 64
BATCH = 16384
NEG = 10

_NC = 2
_NS = 16
_NW = _NC * _NS
_LANES = 16

_BPW = BATCH // _NW
_CH = 32
_NCHUNK = _BPW // _CH
_NGRP = _CH // _LANES

_TBLK = 8192
_NBLK = (VOCAB + _TBLK - 1) // _TBLK
_HB = _TBLK // 2
_PR = _NBLK * _HB
_PW = 2 * EMBED
_SHB = _TBLK.bit_length() - 1
_SHH = _HB.bit_length() - 1


def _sc_scores():
    mesh = plsc.VectorSubcoreMesh(core_axis_name="c", subcore_axis_name="s")

    @functools.partial(
        pl.kernel,
        mesh=mesh,
        compiler_params=pltpu.CompilerParams(needs_layout_passes=False,
                                             use_tc_tiling_on_sc=True),
        out_type=jax.ShapeDtypeStruct(((1 + NEG) * BATCH,), jnp.float32),
        scratch_types=[
            pltpu.VMEM((2 * _CH,), jnp.int32),
            pltpu.VMEM((2 * _CH,), jnp.int32),
            pltpu.VMEM((2 * NEG * _CH,), jnp.int32),
            pltpu.VMEM((2 * _CH,), jnp.int32),
            pltpu.VMEM((2 * _CH,), jnp.int32),
            pltpu.VMEM((2 * NEG * _CH,), jnp.int32),
            pltpu.VMEM((2 * _CH, _PW), jnp.float32),
            pltpu.VMEM((2 * _CH, _PW), jnp.float32),
            pltpu.VMEM((2 * NEG * _CH, _PW), jnp.float32),
            pltpu.VMEM(((1 + NEG) * _CH,), jnp.float32),
            pltpu.SemaphoreType.DMA,
            pltpu.SemaphoreType.DMA,
        ],
    )
    def sc_scores(iw_hbm, cw_hbm, neg_hbm, win_hbm, wctx_hbm, out_hbm,
                  iidx, cidx, nidx, ipar, cpar, npar,
                  irows, crows, nrows, scores, semA, semB):
        wid = lax.axis_index("s") * _NC + lax.axis_index("c")
        lane0 = lax.iota(jnp.int32, _LANES) == 0
        sems = (semA, semB)

        def stage_fire(c):
            p = c & 1
            io, no = p * _CH, p * NEG * _CH
            base = wid * _BPW + c * _CH
            pltpu.sync_copy(iw_hbm.at[pl.ds(base, _CH)],
                            iidx.at[pl.ds(io, _CH)])
            pltpu.sync_copy(cw_hbm.at[pl.ds(base, _CH)],
                            cidx.at[pl.ds(io, _CH)])
            pltpu.sync_copy(neg_hbm.at[pl.ds(base * NEG, NEG * _CH)],
                            nidx.at[pl.ds(no, NEG * _CH)])

            def split_body(t, _):
                off = io + t * _LANES
                for buf, par in ((iidx, ipar), (cidx, cpar)):
                    v = buf[pl.ds(off, _LANES)]
                    par[pl.ds(off, _LANES)] = ((v >> _SHH) & 1) << 6
                    buf[pl.ds(off, _LANES)] = (
                        ((v >> _SHB) << _SHH) | (v & (_HB - 1)))
                return 0

            lax.fori_loop(0, _NGRP, split_body, 0)

            def nsplit_body(t, _):
                off = no + t * _LANES
                v = nidx[pl.ds(off, _LANES)]
                npar[pl.ds(off, _LANES)] = ((v >> _SHH) & 1) << 6
                nidx[pl.ds(off, _LANES)] = (
                    ((v >> _SHB) << _SHH) | (v & (_HB - 1)))
                return 0

            lax.fori_loop(0, NEG * _NGRP, nsplit_body, 0)

            def fire(sem):
                pltpu.async_copy(win_hbm.at[iidx.at[pl.ds(io, _CH)]],
                                 irows.at[pl.ds(io, _CH)], sem)
                pltpu.async_copy(wctx_hbm.at[cidx.at[pl.ds(io, _CH)]],
                                 crows.at[pl.ds(io, _CH)], sem)
                for k in range(NEG):
                    pltpu.async_copy(
                        wctx_hbm.at[nidx.at[pl.ds(no + k * _CH, _CH)]],
                        nrows.at[pl.ds(no + k * _CH, _CH)], sem)

            @pl.when(p == 0)
            def _():
                fire(semA)

            @pl.when(p == 1)
            def _():
                fire(semB)

        def drain(p):
            sem = sems[p]
            io, no = p * _CH, p * NEG * _CH
            pltpu.make_async_copy(win_hbm.at[pl.ds(0, _CH)],
                                  irows.at[pl.ds(io, _CH)], sem).wait()
            pltpu.make_async_copy(win_hbm.at[pl.ds(0, _CH)],
                                  crows.at[pl.ds(io, _CH)], sem).wait()
            pltpu.make_async_copy(win_hbm.at[pl.ds(0, NEG * _CH)],
                                  nrows.at[pl.ds(no, NEG * _CH)], sem).wait()

        def compute_out(c):
            p = c & 1
            io, no = p * _CH, p * NEG * _CH
            base = wid * _BPW + c * _CH

            def group_body(g, _):
                goff = g * _LANES
                pvi = ipar[pl.ds(io + goff, _LANES)]
                pvc = cpar[pl.ds(io + goff, _LANES)]
                pvn = [npar[pl.ds(no + k * _CH + goff, _LANES)]
                       for k in range(NEG)]
                for j16 in range(_LANES):
                    l = goff + j16
                    hi = lax.index_in_dim(pvi, j16, 0, keepdims=False)
                    hc = lax.index_in_dim(pvc, j16, 0, keepdims=False)
                    vin = [irows[io + l, pl.ds(hi + q * _LANES, _LANES)]
                           for q in range(4)]
                    vctx = [crows[io + l, pl.ds(hc + q * _LANES, _LANES)]
                            for q in range(4)]
                    acc = vin[0] * vctx[0]
                    for q in range(1, 4):
                        acc = acc + vin[q] * vctx[q]
                    s = jnp.sum(acc)
                    plsc.store_scatter(
                        scores, [jnp.full((_LANES,), l, jnp.int32)],
                        jnp.full((_LANES,), s, jnp.float32), mask=lane0)
                    for k in range(NEG):
                        hn = lax.index_in_dim(pvn[k], j16, 0, keepdims=False)
                        vng = [nrows[no + k * _CH + l,
                                     pl.ds(hn + q * _LANES, _LANES)]
                               for q in range(4)]
                        nacc = vin[0] * vng[0]
                        for q in range(1, 4):
                            nacc = nacc + vin[q] * vng[q]
                        ns = -jnp.sum(nacc)
                        plsc.store_scatter(
                            scores,
                            [jnp.full((_LANES,), (1 + k) * _CH + l, jnp.int32)],
                            jnp.full((_LANES,), ns, jnp.float32), mask=lane0)
                return 0

            lax.fori_loop(0, _NGRP, group_body, 0)
            for k in range(1 + NEG):
                pltpu.sync_copy(scores.at[pl.ds(k * _CH, _CH)],
                                out_hbm.at[pl.ds(k * BATCH + base, _CH)])

        stage_fire(jnp.int32(0))

        def chunk_body(c, _):
            @pl.when(c + 1 < _NCHUNK)
            def _():
                stage_fire(c + 1)

            @pl.when((c & 1) == 0)
            def _():
                drain(0)

            @pl.when((c & 1) == 1)
            def _():
                drain(1)

            compute_out(c)
            return 0

        lax.fori_loop(0, _NCHUNK, chunk_body, 0)

    return sc_scores


_SC_SCORES = _sc_scores()

def _pack_body(t_ref, o_ref):
    x = t_ref[...]
    x3 = jnp.concatenate([x[:, :_HB], x[:, _HB:]], axis=0)
    eye = jnp.eye(_PW, dtype=jnp.float32)
    o_ref[...] = lax.dot_general(x3, eye, (((0,), (0,)), ((), ())),
                                 preferred_element_type=jnp.float32)


def _to_packed(w_t):
    return pl.pallas_call(
        _pack_body,
        grid=(_NBLK,),
        in_specs=[pl.BlockSpec((EMBED, _TBLK), lambda g: (0, g))],
        out_specs=pl.BlockSpec((_HB, _PW), lambda g: (g, 0)),
        out_shape=jax.ShapeDtypeStruct((_PR, _PW), jnp.float32),
    )(w_t)


_ROWS = (1 + NEG) * BATCH // 128


def _loss_body(s_ref, o_ref):
    x = s_ref[...]
    ls = jnp.minimum(x, 0.0) - jnp.log1p(jnp.exp(-jnp.abs(x)))
    o_ref[0, 0] = -jnp.sum(ls) / BATCH


def kernel(input_word, context_word, W_in, W_ctx):
    batch_size = context_word.shape[0]
    neg_key = jax.random.key(1234)
    negative_example = jax.random.randint(neg_key, (batch_size, NEG), 0, VOCAB)
    neg_cm = (negative_example.astype(jnp.int32)
              .reshape(batch_size // _CH, _CH, NEG)
              .transpose(0, 2, 1)
              .reshape(-1))

    w_in_p = _to_packed(W_in.T)
    w_ctx_p = _to_packed(W_ctx.T)
    scores = _SC_SCORES(input_word.astype(jnp.int32),
                        context_word.astype(jnp.int32),
                        neg_cm, w_in_p, w_ctx_p)

    loss = pl.pallas_call(
        _loss_body,
        out_shape=jax.ShapeDtypeStruct((1, 1), jnp.float32),
        out_specs=pl.BlockSpec(memory_space=pltpu.SMEM),
    )(scores.reshape(_ROWS, 128))
    return loss[0, 0]
